# baseline (speedup 1.0000x reference)
import functools

import jax
import jax.numpy as jnp
import numpy as np
from jax import lax
from jax.experimental import pallas as pl
from jax.experimental.pallas import tpu as pltpu
from jax.experimental.pallas import tpu_sc as plsc

N_PTS = 131072
N_LEVELS = 16
F_PER_LEVEL = 8
NFEAT = N_LEVELS * F_PER_LEVEL
LOG2_T = 16
TABLE_SIZE = 1 << LOG2_T
BASE_RES = 32
PRIME2_I32 = np.int32(2654435761 - (1 << 32))

NC, NS, LANES = 2, 16, 16
NW = NC * NS
PTS_PER_W = N_PTS // NW
G = 128
N_GROUPS = PTS_PER_W // G
SG = G // LANES

COMBOS = ((0, 1), (0, 2), (0, 3), (1, 2), (1, 3), (2, 3))


def _floor_parts(pos):
    pi0 = pos.astype(jnp.int32)
    pif = pi0.astype(jnp.float32)
    neg = pos < pif
    pi = jnp.where(neg, pi0 - 1, pi0)
    pf = jnp.where(neg, pif - 1.0, pif)
    return pi, pos - pf


def _sc_body(coords, tbl, out,
             cbuf, ibuf0, ibuf1, wbuf0, wbuf1, gbuf0, gbuf1, obuf,
             sem0, sem1):
    wid = lax.axis_index("s") * NC + lax.axis_index("c")
    iota = lax.iota(jnp.int32, LANES)

    def hash_stage(l, ibuf, wbuf):
        scale = ((jnp.int32(BASE_RES) << l) - 1).astype(jnp.float32)
        lvl_off = l << LOG2_T

        def body(s, c3):
            off = pl.multiple_of(s * LANES, LANES)
            for p6, (ca, cb2) in enumerate(COMBOS):
                xa = cbuf[ca, pl.ds(off, LANES)]
                xb = cbuf[cb2, pl.ds(off, LANES)]
                posx = xa * scale + 0.5
                posy = xb * scale + 0.5
                pix, wx = _floor_parts(posx)
                piy, wy = _floor_parts(posy)
                a0 = pix
                a1 = pix + 1
                b0 = piy * PRIME2_I32
                b1 = b0 + PRIME2_I32
                p_off = lvl_off + p6 * (N_LEVELS * TABLE_SIZE)
                for c, h in enumerate((a0 ^ b0, a1 ^ b0, a0 ^ b1, a1 ^ b1)):
                    idx = (h & jnp.int32(TABLE_SIZE - 1)) + p_off
                    ibuf[pl.ds((p6 * 4 + c) * G + off, LANES)] = idx
                wbuf[p6 * 2, pl.ds(off, LANES)] = wx
                wbuf[p6 * 2 + 1, pl.ds(off, LANES)] = wy
            return c3

        lax.fori_loop(0, SG, body, 0)

    def copies(ibuf, gbuf, sem):
        return [pltpu.make_async_copy(tbl.at[ibuf], gbuf, sem)]

    def fire(ibuf, gbuf, sem):
        for cp in copies(ibuf, gbuf, sem):
            cp.start()

    def drain(ibuf, gbuf, sem):
        for cp in copies(ibuf, gbuf, sem):
            cp.wait()

    def mac_stage(l, wbuf, gbuf):
        def body(s, c3):
            soff = pl.multiple_of(s * LANES, LANES)
            acc = [None] * F_PER_LEVEL
            for p6 in range(6):
                wx = wbuf[p6 * 2, pl.ds(soff, LANES)]
                wy = wbuf[p6 * 2 + 1, pl.ds(soff, LANES)]
                rows = [iota + (p6 * 4 + c) * G + soff for c in range(4)]
                for f in range(F_PER_LEVEL):
                    col = jnp.full((LANES,), f, jnp.int32)
                    r00 = plsc.load_gather(gbuf, [rows[0], col])
                    r10 = plsc.load_gather(gbuf, [rows[1], col])
                    r01 = plsc.load_gather(gbuf, [rows[2], col])
                    r11 = plsc.load_gather(gbuf, [rows[3], col])
                    r0 = r00 + wx * (r10 - r00)
                    r1 = r01 + wx * (r11 - r01)
                    v = r0 + wy * (r1 - r0)
                    acc[f] = v if p6 == 0 else acc[f] * v
            for f in range(F_PER_LEVEL):
                plsc.store_scatter(
                    obuf, [iota + soff,
                           jnp.full((LANES,), f, jnp.int32) + l * F_PER_LEVEL],
                    acc[f])
            return c3

        lax.fori_loop(0, SG, body, 0)

    def group_body(g, carry):
        base = wid * PTS_PER_W + g * G
        base = pl.multiple_of(base, G)
        pltpu.sync_copy(coords.at[:, pl.ds(base, G)], cbuf)

        hash_stage(0, ibuf0, wbuf0)
        fire(ibuf0, gbuf0, sem0)

        def it_body(it, carry2):
            la = 2 * it
            hash_stage(la + 1, ibuf1, wbuf1)
            fire(ibuf1, gbuf1, sem1)
            drain(ibuf0, gbuf0, sem0)
            mac_stage(la, wbuf0, gbuf0)

            @pl.when(it < N_LEVELS // 2 - 1)
            def _():
                hash_stage(la + 2, ibuf0, wbuf0)
                fire(ibuf0, gbuf0, sem0)

            drain(ibuf1, gbuf1, sem1)
            mac_stage(la + 1, wbuf1, gbuf1)
            return carry2

        lax.fori_loop(0, N_LEVELS // 2, it_body, 0)
        pltpu.sync_copy(obuf, out.at[pl.ds(base, G)])
        return carry

    lax.fori_loop(0, N_GROUPS, group_body, 0)


TPW = 8192 // NW
KP = 16


def _detile_body(v0, v1, v2, v3, v4, v5, out,
                 vbuf0, vbuf1, obuf0, obuf1, isem0, isem1, osem0, osem1):
    wid = lax.axis_index("s") * NC + lax.axis_index("c")
    iota = lax.iota(jnp.int32, LANES)
    fpat = iota & 7
    jpat = lax.shift_right_logical(iota, 3)
    vbufs, obufs = (vbuf0, vbuf1), (obuf0, obuf1)
    isems, osems = (isem0, isem1), (osem0, osem1)
    tables = (v0, v1, v2, v3, v4, v5)
    NCH = TPW // KP

    def transpose(b):
        def panel_body(k, c2):
            kb = pl.multiple_of(k * 128, 128)
            kv = jnp.full((LANES,), k, jnp.int32)
            def pair_body(j8, c3):
                for u in range(8):
                    jv = jpat + (2 * 8) * j8 + 2 * u
                    x = plsc.load_gather(vbufs[b], [kv, fpat, jv])
                    plsc.store_scatter(
                        obufs[b], [kb + (2 * 8) * j8 + 2 * u + jpat, fpat], x)
                return c3
            lax.fori_loop(0, 8, pair_body, 0)
            return c2
        lax.fori_loop(0, KP, panel_body, 0)

    for ti, v in enumerate(tables):
        def in_cp(c, b, v=v):
            t0 = pl.multiple_of(wid * TPW + c * KP, KP)
            return pltpu.make_async_copy(v.at[pl.ds(t0, KP)], vbufs[b],
                                         isems[b])

        def out_cp(c, b, ti=ti):
            t0 = pl.multiple_of(wid * TPW + c * KP, KP)
            dst = out.at[pl.ds(ti * (N_LEVELS * TABLE_SIZE) + t0 * 128,
                               KP * 128)]
            return pltpu.make_async_copy(obufs[b], dst, osems[b])

        in_cp(0, 0).start()

        def dbody(it, carry, in_cp=in_cp, out_cp=out_cp):
            ca = 2 * it
            in_cp(ca + 1, 1).start()
            in_cp(ca, 0).wait()

            @pl.when(it > 0)
            def _():
                out_cp(ca - 2, 0).wait()

            transpose(0)
            out_cp(ca, 0).start()

            @pl.when(it < NCH // 2 - 1)
            def _():
                in_cp(ca + 2, 0).start()

            in_cp(ca + 1, 1).wait()

            @pl.when(it > 0)
            def _():
                out_cp(ca - 1, 1).wait()

            transpose(1)
            out_cp(ca + 1, 1).start()
            return carry

        lax.fori_loop(0, NCH // 2, dbody, 0)
        out_cp(NCH - 2, 0).wait()
        out_cp(NCH - 1, 1).wait()


@jax.jit
def _detile(v0, v1, v2, v3, v4, v5):
    mesh = plsc.VectorSubcoreMesh(core_axis_name="c", subcore_axis_name="s")
    fn = functools.partial(
        pl.kernel,
        mesh=mesh,
        out_type=jax.ShapeDtypeStruct((6 * N_LEVELS * TABLE_SIZE, F_PER_LEVEL),
                                      jnp.float32),
        scratch_types=[
            pltpu.VMEM((KP, F_PER_LEVEL, 128), jnp.float32),
            pltpu.VMEM((KP, F_PER_LEVEL, 128), jnp.float32),
            pltpu.VMEM((KP * 128, F_PER_LEVEL), jnp.float32),
            pltpu.VMEM((KP * 128, F_PER_LEVEL), jnp.float32),
            pltpu.SemaphoreType.DMA,
            pltpu.SemaphoreType.DMA,
            pltpu.SemaphoreType.DMA,
            pltpu.SemaphoreType.DMA,
        ],
        compiler_params=pltpu.CompilerParams(
            use_tc_tiling_on_sc=False, needs_layout_passes=False),
    )(_detile_body)
    return fn(v0, v1, v2, v3, v4, v5)


@jax.jit
def _encode(coords, tbl):
    mesh = plsc.VectorSubcoreMesh(core_axis_name="c", subcore_axis_name="s")
    fn = functools.partial(
        pl.kernel,
        mesh=mesh,
        out_type=jax.ShapeDtypeStruct((N_PTS, NFEAT), jnp.float32),
        scratch_types=[
            pltpu.VMEM((4, G), jnp.float32),
            pltpu.VMEM((24 * G,), jnp.int32),
            pltpu.VMEM((24 * G,), jnp.int32),
            pltpu.VMEM((12, G), jnp.float32),
            pltpu.VMEM((12, G), jnp.float32),
            pltpu.VMEM((24 * G, F_PER_LEVEL), jnp.float32),
            pltpu.VMEM((24 * G, F_PER_LEVEL), jnp.float32),
            pltpu.VMEM((G, NFEAT), jnp.float32),
            pltpu.SemaphoreType.DMA,
            pltpu.SemaphoreType.DMA,
        ],
        compiler_params=pltpu.CompilerParams(
            use_tc_tiling_on_sc=False, needs_layout_passes=False),
    )(_sc_body)
    return fn(coords, tbl)


def kernel(pts, timestamps, aabb, table0, table1, table2, table3, table4, table5):
    pts_n = (pts - aabb[0]) * (2.0 / (aabb[1] - aabb[0])) - 1.0
    p4 = jnp.concatenate([pts_n, timestamps], axis=-1)
    coords = p4.T
    views = [t.T.reshape(8, 8192, 128).transpose(1, 0, 2)
             for t in (table0, table1, table2, table3, table4, table5)]
    tbl = _detile(*views)
    return _encode(coords, tbl)


# speedup vs baseline: 1.1385x; 1.1385x over previous
import functools

import jax
import jax.numpy as jnp
import numpy as np
from jax import lax
from jax.experimental import pallas as pl
from jax.experimental.pallas import tpu as pltpu
from jax.experimental.pallas import tpu_sc as plsc

N_PTS = 131072
N_LEVELS = 16
F_PER_LEVEL = 8
NFEAT = N_LEVELS * F_PER_LEVEL
LOG2_T = 16
TABLE_SIZE = 1 << LOG2_T
BASE_RES = 32
PRIME2_I32 = np.int32(2654435761 - (1 << 32))

NC, NS, LANES = 2, 16, 16
NW = NC * NS
PTS_PER_W = N_PTS // NW
G = 128
N_GROUPS = PTS_PER_W // G
SG = G // LANES

COMBOS = ((0, 1), (0, 2), (0, 3), (1, 2), (1, 3), (2, 3))


def _floor_parts(pos):
    pi0 = pos.astype(jnp.int32)
    pif = pi0.astype(jnp.float32)
    neg = pos < pif
    pi = jnp.where(neg, pi0 - 1, pi0)
    pf = jnp.where(neg, pif - 1.0, pif)
    return pi, pos - pf


def _sc_body(coords, tbl, out,
             cbuf, ibuf0, ibuf1, wbuf0, wbuf1, gbuf0, gbuf1, obuf,
             sem0, sem1):
    wid = lax.axis_index("s") * NC + lax.axis_index("c")
    iota = lax.iota(jnp.int32, LANES)

    def hash_stage(l, ibuf, wbuf):
        scale = ((jnp.int32(BASE_RES) << l) - 1).astype(jnp.float32)
        lvl_off = l << LOG2_T

        def body(s, c3):
            off = pl.multiple_of(s * LANES, LANES)
            for p6, (ca, cb2) in enumerate(COMBOS):
                xa = cbuf[ca, pl.ds(off, LANES)]
                xb = cbuf[cb2, pl.ds(off, LANES)]
                posx = xa * scale + 0.5
                posy = xb * scale + 0.5
                pix, wx = _floor_parts(posx)
                piy, wy = _floor_parts(posy)
                a0 = pix
                a1 = pix + 1
                b0 = piy * PRIME2_I32
                b1 = b0 + PRIME2_I32
                p_off = lvl_off + p6 * (N_LEVELS * TABLE_SIZE)
                for c, h in enumerate((a0 ^ b0, a1 ^ b0, a0 ^ b1, a1 ^ b1)):
                    idx = (h & jnp.int32(TABLE_SIZE - 1)) + p_off
                    ibuf[pl.ds((p6 * 4 + c) * G + off, LANES)] = idx
                wbuf[p6 * 2, pl.ds(off, LANES)] = wx
                wbuf[p6 * 2 + 1, pl.ds(off, LANES)] = wy
            return c3

        lax.fori_loop(0, SG, body, 0)

    def copies(ibuf, gbuf, sem):
        return [pltpu.make_async_copy(tbl.at[ibuf], gbuf, sem)]

    def fire(ibuf, gbuf, sem):
        for cp in copies(ibuf, gbuf, sem):
            cp.start()

    def drain(ibuf, gbuf, sem):
        for cp in copies(ibuf, gbuf, sem):
            cp.wait()

    def mac_stage(l, wbuf, gbuf):
        def body(s, c3):
            soff = pl.multiple_of(s * LANES, LANES)
            acc = [None] * F_PER_LEVEL
            for p6 in range(6):
                wx = wbuf[p6 * 2, pl.ds(soff, LANES)]
                wy = wbuf[p6 * 2 + 1, pl.ds(soff, LANES)]
                rows = [iota + (p6 * 4 + c) * G + soff for c in range(4)]
                for f in range(F_PER_LEVEL):
                    col = jnp.full((LANES,), f, jnp.int32)
                    r00 = plsc.load_gather(gbuf, [rows[0], col])
                    r10 = plsc.load_gather(gbuf, [rows[1], col])
                    r01 = plsc.load_gather(gbuf, [rows[2], col])
                    r11 = plsc.load_gather(gbuf, [rows[3], col])
                    r0 = r00 + wx * (r10 - r00)
                    r1 = r01 + wx * (r11 - r01)
                    v = r0 + wy * (r1 - r0)
                    acc[f] = v if p6 == 0 else acc[f] * v
            for f in range(F_PER_LEVEL):
                plsc.store_scatter(
                    obuf, [iota + soff,
                           jnp.full((LANES,), f, jnp.int32) + l * F_PER_LEVEL],
                    acc[f])
            return c3

        lax.fori_loop(0, SG, body, 0)

    def group_body(g, carry):
        base = wid * PTS_PER_W + g * G
        base = pl.multiple_of(base, G)
        pltpu.sync_copy(coords.at[:, pl.ds(base, G)], cbuf)

        hash_stage(0, ibuf0, wbuf0)
        fire(ibuf0, gbuf0, sem0)

        def it_body(it, carry2):
            la = 2 * it
            hash_stage(la + 1, ibuf1, wbuf1)
            fire(ibuf1, gbuf1, sem1)
            drain(ibuf0, gbuf0, sem0)
            mac_stage(la, wbuf0, gbuf0)

            @pl.when(it < N_LEVELS // 2 - 1)
            def _():
                hash_stage(la + 2, ibuf0, wbuf0)
                fire(ibuf0, gbuf0, sem0)

            drain(ibuf1, gbuf1, sem1)
            mac_stage(la + 1, wbuf1, gbuf1)
            return carry2

        lax.fori_loop(0, N_LEVELS // 2, it_body, 0)
        pltpu.sync_copy(obuf, out.at[pl.ds(base, G)])
        return carry

    lax.fori_loop(0, N_GROUPS, group_body, 0)


TPW = 8192 // NW
KP = 16


def _detile_body(v0, v1, v2, v3, v4, v5, out,
                 vbuf0, vbuf1, obuf0, obuf1, isem0, isem1, osem0, osem1):
    wid = lax.axis_index("s") * NC + lax.axis_index("c")
    iota = lax.iota(jnp.int32, LANES)
    fpat = iota & 7
    jpat = lax.shift_right_logical(iota, 3)
    vbufs, obufs = (vbuf0, vbuf1), (obuf0, obuf1)
    isems, osems = (isem0, isem1), (osem0, osem1)
    tables = (v0, v1, v2, v3, v4, v5)
    NCH = TPW // KP

    skew = jpat + 2 * fpat

    def transpose(b):
        def panel_body(k, c2):
            kb = pl.multiple_of(k * 128, 128)
            kv = jnp.full((LANES,), k, jnp.int32)
            def win_body(w, c3):
                wj = pl.multiple_of(w * 16, 16)
                for j0 in range(0, 16, 2):
                    jv = wj + ((skew + j0) & 15)
                    x = plsc.load_gather(vbufs[b], [kv, fpat, jv])
                    plsc.store_scatter(obufs[b], [kb + jv, fpat], x)
                return c3
            lax.fori_loop(0, 8, win_body, 0)
            return c2
        lax.fori_loop(0, KP, panel_body, 0)

    for ti, v in enumerate(tables):
        def in_cp(c, b, v=v):
            t0 = pl.multiple_of(wid * TPW + c * KP, KP)
            return pltpu.make_async_copy(v.at[pl.ds(t0, KP)], vbufs[b],
                                         isems[b])

        def out_cp(c, b, ti=ti):
            t0 = pl.multiple_of(wid * TPW + c * KP, KP)
            dst = out.at[pl.ds(ti * (N_LEVELS * TABLE_SIZE) + t0 * 128,
                               KP * 128)]
            return pltpu.make_async_copy(obufs[b], dst, osems[b])

        in_cp(0, 0).start()

        def dbody(it, carry, in_cp=in_cp, out_cp=out_cp):
            ca = 2 * it
            in_cp(ca + 1, 1).start()
            in_cp(ca, 0).wait()

            @pl.when(it > 0)
            def _():
                out_cp(ca - 2, 0).wait()

            transpose(0)
            out_cp(ca, 0).start()

            @pl.when(it < NCH // 2 - 1)
            def _():
                in_cp(ca + 2, 0).start()

            in_cp(ca + 1, 1).wait()

            @pl.when(it > 0)
            def _():
                out_cp(ca - 1, 1).wait()

            transpose(1)
            out_cp(ca + 1, 1).start()
            return carry

        lax.fori_loop(0, NCH // 2, dbody, 0)
        out_cp(NCH - 2, 0).wait()
        out_cp(NCH - 1, 1).wait()


@jax.jit
def _detile(v0, v1, v2, v3, v4, v5):
    mesh = plsc.VectorSubcoreMesh(core_axis_name="c", subcore_axis_name="s")
    fn = functools.partial(
        pl.kernel,
        mesh=mesh,
        out_type=jax.ShapeDtypeStruct((6 * N_LEVELS * TABLE_SIZE, F_PER_LEVEL),
                                      jnp.float32),
        scratch_types=[
            pltpu.VMEM((KP, F_PER_LEVEL, 128), jnp.float32),
            pltpu.VMEM((KP, F_PER_LEVEL, 128), jnp.float32),
            pltpu.VMEM((KP * 128, F_PER_LEVEL), jnp.float32),
            pltpu.VMEM((KP * 128, F_PER_LEVEL), jnp.float32),
            pltpu.SemaphoreType.DMA,
            pltpu.SemaphoreType.DMA,
            pltpu.SemaphoreType.DMA,
            pltpu.SemaphoreType.DMA,
        ],
        compiler_params=pltpu.CompilerParams(
            use_tc_tiling_on_sc=False, needs_layout_passes=False),
    )(_detile_body)
    return fn(v0, v1, v2, v3, v4, v5)


@jax.jit
def _encode(coords, tbl):
    mesh = plsc.VectorSubcoreMesh(core_axis_name="c", subcore_axis_name="s")
    fn = functools.partial(
        pl.kernel,
        mesh=mesh,
        out_type=jax.ShapeDtypeStruct((N_PTS, NFEAT), jnp.float32),
        scratch_types=[
            pltpu.VMEM((4, G), jnp.float32),
            pltpu.VMEM((24 * G,), jnp.int32),
            pltpu.VMEM((24 * G,), jnp.int32),
            pltpu.VMEM((12, G), jnp.float32),
            pltpu.VMEM((12, G), jnp.float32),
            pltpu.VMEM((24 * G, F_PER_LEVEL), jnp.float32),
            pltpu.VMEM((24 * G, F_PER_LEVEL), jnp.float32),
            pltpu.VMEM((G, NFEAT), jnp.float32),
            pltpu.SemaphoreType.DMA,
            pltpu.SemaphoreType.DMA,
        ],
        compiler_params=pltpu.CompilerParams(
            use_tc_tiling_on_sc=False, needs_layout_passes=False),
    )(_sc_body)
    return fn(coords, tbl)


def kernel(pts, timestamps, aabb, table0, table1, table2, table3, table4, table5):
    pts_n = (pts - aabb[0]) * (2.0 / (aabb[1] - aabb[0])) - 1.0
    p4 = jnp.concatenate([pts_n, timestamps], axis=-1)
    coords = p4.T
    views = [t.T.reshape(8, 8192, 128).transpose(1, 0, 2)
             for t in (table0, table1, table2, table3, table4, table5)]
    tbl = _detile(*views)
    return _encode(coords, tbl)


# speedup vs baseline: 1.1810x; 1.0373x over previous
import functools

import jax
import jax.numpy as jnp
import numpy as np
from jax import lax
from jax.experimental import pallas as pl
from jax.experimental.pallas import tpu as pltpu
from jax.experimental.pallas import tpu_sc as plsc

N_PTS = 131072
N_LEVELS = 16
F_PER_LEVEL = 8
NFEAT = N_LEVELS * F_PER_LEVEL
LOG2_T = 16
TABLE_SIZE = 1 << LOG2_T
BASE_RES = 32
PRIME2_I32 = np.int32(2654435761 - (1 << 32))

NC, NS, LANES = 2, 16, 16
NW = NC * NS
PTS_PER_W = N_PTS // NW
G = 128
N_GROUPS = PTS_PER_W // G
SG = G // LANES

COMBOS = ((0, 1), (0, 2), (0, 3), (1, 2), (1, 3), (2, 3))


def _take16(v, idx):
    dnums = lax.GatherDimensionNumbers(
        offset_dims=(), collapsed_slice_dims=(0,), start_index_map=(0,))
    return lax.gather(v, idx[:, None], dnums, slice_sizes=(1,),
                      mode=lax.GatherScatterMode.PROMISE_IN_BOUNDS)


def _floor_parts(pos):
    pi0 = pos.astype(jnp.int32)
    pif = pi0.astype(jnp.float32)
    neg = pos < pif
    pi = jnp.where(neg, pi0 - 1, pi0)
    pf = jnp.where(neg, pif - 1.0, pif)
    return pi, pos - pf


def _sc_body(coords, tbl, out,
             cbuf, ibuf0, ibuf1, wbuf0, wbuf1, gbuf0, gbuf1, obuf,
             sem0, sem1):
    wid = lax.axis_index("s") * NC + lax.axis_index("c")
    iota = lax.iota(jnp.int32, LANES)

    def hash_stage(l, ibuf, wbuf):
        scale = ((jnp.int32(BASE_RES) << l) - 1).astype(jnp.float32)
        lvl_off = l << LOG2_T

        def body(s, c3):
            off = pl.multiple_of(s * LANES, LANES)
            for p6, (ca, cb2) in enumerate(COMBOS):
                xa = cbuf[ca, pl.ds(off, LANES)]
                xb = cbuf[cb2, pl.ds(off, LANES)]
                posx = xa * scale + 0.5
                posy = xb * scale + 0.5
                pix, wx = _floor_parts(posx)
                piy, wy = _floor_parts(posy)
                a0 = pix
                a1 = pix + 1
                b0 = piy * PRIME2_I32
                b1 = b0 + PRIME2_I32
                p_off = lvl_off + p6 * (N_LEVELS * TABLE_SIZE)
                for c, h in enumerate((a0 ^ b0, a1 ^ b0, a0 ^ b1, a1 ^ b1)):
                    idx = (h & jnp.int32(TABLE_SIZE - 1)) + p_off
                    ibuf[pl.ds((p6 * 4 + c) * G + off, LANES)] = idx
                wbuf[p6 * 2, pl.ds(off, LANES)] = wx
                wbuf[p6 * 2 + 1, pl.ds(off, LANES)] = wy
            return c3

        lax.fori_loop(0, SG, body, 0)

    def copies(ibuf, gbuf, sem):
        return [pltpu.make_async_copy(tbl.at[ibuf], gbuf, sem)]

    def fire(ibuf, gbuf, sem):
        for cp in copies(ibuf, gbuf, sem):
            cp.start()

    def drain(ibuf, gbuf, sem):
        for cp in copies(ibuf, gbuf, sem):
            cp.wait()

    fpat = iota & 7
    upat = lax.shift_right_logical(iota, 3)

    def mac_stage(l, wbuf, gbuf):
        def body(s, c3):
            soff = pl.multiple_of(s * LANES, LANES)
            acc = [None] * (LANES // 2)
            for p6 in range(6):
                wx = wbuf[p6 * 2, pl.ds(soff, LANES)]
                wy = wbuf[p6 * 2 + 1, pl.ds(soff, LANES)]
                prow = soff + upat + p6 * 4 * G
                for q in range(LANES // 2):
                    widx = upat + 2 * q
                    wxe = _take16(wx, widx)
                    wye = _take16(wy, widx)
                    rq = prow + 2 * q
                    r00 = plsc.load_gather(gbuf, [rq, fpat])
                    r10 = plsc.load_gather(gbuf, [rq + G, fpat])
                    r01 = plsc.load_gather(gbuf, [rq + 2 * G, fpat])
                    r11 = plsc.load_gather(gbuf, [rq + 3 * G, fpat])
                    r0 = r00 + wxe * (r10 - r00)
                    r1 = r01 + wxe * (r11 - r01)
                    v = r0 + wye * (r1 - r0)
                    acc[q] = v if p6 == 0 else acc[q] * v
            for q in range(LANES // 2):
                plsc.store_scatter(
                    obuf, [soff + 2 * q + upat, fpat + l * F_PER_LEVEL],
                    acc[q])
            return c3

        lax.fori_loop(0, SG, body, 0)

    def group_body(g, carry):
        base = wid * PTS_PER_W + g * G
        base = pl.multiple_of(base, G)
        pltpu.sync_copy(coords.at[:, pl.ds(base, G)], cbuf)

        hash_stage(0, ibuf0, wbuf0)
        fire(ibuf0, gbuf0, sem0)

        def it_body(it, carry2):
            la = 2 * it
            hash_stage(la + 1, ibuf1, wbuf1)
            fire(ibuf1, gbuf1, sem1)
            drain(ibuf0, gbuf0, sem0)
            mac_stage(la, wbuf0, gbuf0)

            @pl.when(it < N_LEVELS // 2 - 1)
            def _():
                hash_stage(la + 2, ibuf0, wbuf0)
                fire(ibuf0, gbuf0, sem0)

            drain(ibuf1, gbuf1, sem1)
            mac_stage(la + 1, wbuf1, gbuf1)
            return carry2

        lax.fori_loop(0, N_LEVELS // 2, it_body, 0)
        pltpu.sync_copy(obuf, out.at[pl.ds(base, G)])
        return carry

    lax.fori_loop(0, N_GROUPS, group_body, 0)


TPW = 8192 // NW
KP = 16


def _detile_body(v0, v1, v2, v3, v4, v5, out,
                 vbuf0, vbuf1, obuf0, obuf1, isem0, isem1, osem0, osem1):
    wid = lax.axis_index("s") * NC + lax.axis_index("c")
    iota = lax.iota(jnp.int32, LANES)
    fpat = iota & 7
    jpat = lax.shift_right_logical(iota, 3)
    vbufs, obufs = (vbuf0, vbuf1), (obuf0, obuf1)
    isems, osems = (isem0, isem1), (osem0, osem1)
    tables = (v0, v1, v2, v3, v4, v5)
    NCH = TPW // KP

    skew = jpat + 2 * fpat

    def transpose(b):
        def panel_body(k, c2):
            kb = pl.multiple_of(k * 128, 128)
            kv = jnp.full((LANES,), k, jnp.int32)
            def win_body(w, c3):
                wj = pl.multiple_of(w * 16, 16)
                for j0 in range(0, 16, 2):
                    jv = wj + ((skew + j0) & 15)
                    x = plsc.load_gather(vbufs[b], [kv, fpat, jv])
                    plsc.store_scatter(obufs[b], [kb + jv, fpat], x)
                return c3
            lax.fori_loop(0, 8, win_body, 0)
            return c2
        lax.fori_loop(0, KP, panel_body, 0)

    for ti, v in enumerate(tables):
        def in_cp(c, b, v=v):
            t0 = pl.multiple_of(wid * TPW + c * KP, KP)
            return pltpu.make_async_copy(v.at[pl.ds(t0, KP)], vbufs[b],
                                         isems[b])

        def out_cp(c, b, ti=ti):
            t0 = pl.multiple_of(wid * TPW + c * KP, KP)
            dst = out.at[pl.ds(ti * (N_LEVELS * TABLE_SIZE) + t0 * 128,
                               KP * 128)]
            return pltpu.make_async_copy(obufs[b], dst, osems[b])

        in_cp(0, 0).start()

        def dbody(it, carry, in_cp=in_cp, out_cp=out_cp):
            ca = 2 * it
            in_cp(ca + 1, 1).start()
            in_cp(ca, 0).wait()

            @pl.when(it > 0)
            def _():
                out_cp(ca - 2, 0).wait()

            transpose(0)
            out_cp(ca, 0).start()

            @pl.when(it < NCH // 2 - 1)
            def _():
                in_cp(ca + 2, 0).start()

            in_cp(ca + 1, 1).wait()

            @pl.when(it > 0)
            def _():
                out_cp(ca - 1, 1).wait()

            transpose(1)
            out_cp(ca + 1, 1).start()
            return carry

        lax.fori_loop(0, NCH // 2, dbody, 0)
        out_cp(NCH - 2, 0).wait()
        out_cp(NCH - 1, 1).wait()


@jax.jit
def _detile(v0, v1, v2, v3, v4, v5):
    mesh = plsc.VectorSubcoreMesh(core_axis_name="c", subcore_axis_name="s")
    fn = functools.partial(
        pl.kernel,
        mesh=mesh,
        out_type=jax.ShapeDtypeStruct((6 * N_LEVELS * TABLE_SIZE, F_PER_LEVEL),
                                      jnp.float32),
        scratch_types=[
            pltpu.VMEM((KP, F_PER_LEVEL, 128), jnp.float32),
            pltpu.VMEM((KP, F_PER_LEVEL, 128), jnp.float32),
            pltpu.VMEM((KP * 128, F_PER_LEVEL), jnp.float32),
            pltpu.VMEM((KP * 128, F_PER_LEVEL), jnp.float32),
            pltpu.SemaphoreType.DMA,
            pltpu.SemaphoreType.DMA,
            pltpu.SemaphoreType.DMA,
            pltpu.SemaphoreType.DMA,
        ],
        compiler_params=pltpu.CompilerParams(
            use_tc_tiling_on_sc=False, needs_layout_passes=False),
    )(_detile_body)
    return fn(v0, v1, v2, v3, v4, v5)


@jax.jit
def _encode(coords, tbl):
    mesh = plsc.VectorSubcoreMesh(core_axis_name="c", subcore_axis_name="s")
    fn = functools.partial(
        pl.kernel,
        mesh=mesh,
        out_type=jax.ShapeDtypeStruct((N_PTS, NFEAT), jnp.float32),
        scratch_types=[
            pltpu.VMEM((4, G), jnp.float32),
            pltpu.VMEM((24 * G,), jnp.int32),
            pltpu.VMEM((24 * G,), jnp.int32),
            pltpu.VMEM((12, G), jnp.float32),
            pltpu.VMEM((12, G), jnp.float32),
            pltpu.VMEM((24 * G, F_PER_LEVEL), jnp.float32),
            pltpu.VMEM((24 * G, F_PER_LEVEL), jnp.float32),
            pltpu.VMEM((G, NFEAT), jnp.float32),
            pltpu.SemaphoreType.DMA,
            pltpu.SemaphoreType.DMA,
        ],
        compiler_params=pltpu.CompilerParams(
            use_tc_tiling_on_sc=False, needs_layout_passes=False),
    )(_sc_body)
    return fn(coords, tbl)


def kernel(pts, timestamps, aabb, table0, table1, table2, table3, table4, table5):
    pts_n = (pts - aabb[0]) * (2.0 / (aabb[1] - aabb[0])) - 1.0
    p4 = jnp.concatenate([pts_n, timestamps], axis=-1)
    coords = p4.T
    views = [t.T.reshape(8, 8192, 128).transpose(1, 0, 2)
             for t in (table0, table1, table2, table3, table4, table5)]
    tbl = _detile(*views)
    return _encode(coords, tbl)
